# baseline (speedup 1.0000x reference)
import functools

import jax
import jax.numpy as jnp
from jax import lax
from jax.experimental import pallas as pl
from jax.experimental.pallas import tpu as pltpu
from jax.experimental.pallas import tpu_sc as plsc


def _largest_chunk(k: int) -> int:
    for c in range(min(k, 128), 0, -1):
        if k % c == 0 and c % 16 == 0:
            return c
    return 16


@functools.lru_cache(maxsize=None)
def _make_gather(N: int, C: int, K: int, D: int):
    info = plsc.get_sparse_core_info()
    NC, NS = info.num_cores, info.num_subcores
    NW = NC * NS
    BPW = N // NW
    CHUNK = _largest_chunk(K)
    NCH = K // CHUNK
    mesh = plsc.VectorSubcoreMesh(core_axis_name="c", subcore_axis_name="s")

    @functools.partial(
        pl.kernel,
        mesh=mesh,
        compiler_params=pltpu.CompilerParams(use_tc_tiling_on_sc=False),
        out_type=jax.ShapeDtypeStruct((N * K, D), jnp.float32),
        scratch_types=[
            pltpu.VMEM((K,), jnp.int32),
            pltpu.VMEM((NCH, CHUNK), jnp.int32),
            pltpu.VMEM((CHUNK, D), jnp.float32),
            pltpu.SemaphoreType.DMA,
        ],
    )
    def gather_kernel(table_hbm, idx_hbm, out_hbm, idx_raw, idx_rows, buf, sem):
        wid = lax.axis_index("s") * NC + lax.axis_index("c")
        pltpu.sync_copy(idx_hbm, idx_raw)
        n0 = wid * BPW
        for b in range(BPW):
            n = n0 + b
            row_base = n * C
            for ch in range(NCH):
                for i in range(CHUNK // 16):
                    src = pl.ds(ch * CHUNK + i * 16, 16)
                    dst = pl.ds(i * 16, 16)
                    idx_rows[ch, dst] = idx_raw[src] + row_base
            for ch in range(NCH):
                pltpu.async_copy(table_hbm.at[idx_rows.at[ch]], buf, sem).wait()
                out0 = n * K + ch * CHUNK
                pltpu.sync_copy(buf, out_hbm.at[pl.ds(out0, CHUNK)])

    return gather_kernel


def kernel(input_tensor, indexes):
    N, C, H, W = input_tensor.shape
    K = indexes.shape[0]
    if K == C:
        return input_tensor
    D = H * W
    table = input_tensor.reshape(N * C, D)
    out = _make_gather(N, C, K, D)(table, indexes)
    return out.reshape(N, K, H, W)


# speedup vs baseline: 1.0003x; 1.0003x over previous
import functools

import jax
import jax.numpy as jnp
from jax import lax
from jax.experimental import pallas as pl
from jax.experimental.pallas import tpu as pltpu
from jax.experimental.pallas import tpu_sc as plsc


def _largest_chunk(k: int) -> int:
    for c in range(min(k, 128), 0, -1):
        if k % c == 0 and c % 16 == 0:
            return c
    return 16


@functools.lru_cache(maxsize=None)
def _make_gather(N: int, C: int, K: int, D: int):
    info = plsc.get_sparse_core_info()
    NC, NS = info.num_cores, info.num_subcores
    NW = NC * NS
    BPW = N // NW
    CHUNK = _largest_chunk(K)
    NCH = K // CHUNK
    mesh = plsc.VectorSubcoreMesh(core_axis_name="c", subcore_axis_name="s")

    T = BPW * NCH

    @functools.partial(
        pl.kernel,
        mesh=mesh,
        compiler_params=pltpu.CompilerParams(use_tc_tiling_on_sc=False),
        out_type=jax.ShapeDtypeStruct((N * K, D), jnp.float32),
        scratch_types=[
            pltpu.VMEM((K,), jnp.int32),
            pltpu.VMEM((T, CHUNK), jnp.int32),
            pltpu.VMEM((2, CHUNK, D), jnp.float32),
            pltpu.SemaphoreType.DMA,
            pltpu.SemaphoreType.DMA,
            pltpu.SemaphoreType.DMA,
            pltpu.SemaphoreType.DMA,
        ],
    )
    def gather_kernel(table_hbm, idx_hbm, out_hbm, idx_raw, idx_rows,
                      buf, sg0, sg1, sw0, sw1):
        sg = (sg0, sg1)
        sw = (sw0, sw1)
        wid = lax.axis_index("s") * NC + lax.axis_index("c")
        pltpu.sync_copy(idx_hbm, idx_raw)
        n0 = wid * BPW

        def build_idx(t):
            b, ch = divmod(t, NCH)
            row_base = (n0 + b) * C
            for i in range(CHUNK // 16):
                src = pl.ds(ch * CHUNK + i * 16, 16)
                dst = pl.ds(i * 16, 16)
                idx_rows[t, dst] = idx_raw[src] + row_base

        def gather_start(t):
            return pltpu.async_copy(
                table_hbm.at[idx_rows.at[t]], buf.at[t % 2], sg[t % 2])

        def write_start(t):
            b, ch = divmod(t, NCH)
            out0 = (n0 + b) * K + ch * CHUNK
            return pltpu.async_copy(
                buf.at[t % 2], out_hbm.at[pl.ds(out0, CHUNK)], sw[t % 2])

        build_idx(0)
        g = [None] * T
        w = [None] * T
        g[0] = gather_start(0)
        for t in range(1, T):
            build_idx(t)
        for t in range(T):
            g[t].wait()
            if t + 1 < T:
                if t >= 1:
                    w[t - 1].wait()
                g[t + 1] = gather_start(t + 1)
            w[t] = write_start(t)
        if T >= 2:
            w[T - 2].wait()
        w[T - 1].wait()

    return gather_kernel


def kernel(input_tensor, indexes):
    N, C, H, W = input_tensor.shape
    K = indexes.shape[0]
    if K == C:
        return input_tensor
    D = H * W
    table = input_tensor.reshape(N * C, D)
    out = _make_gather(N, C, K, D)(table, indexes)
    return out.reshape(N, K, H, W)


# speedup vs baseline: 3.0090x; 3.0082x over previous
import functools

import jax
import jax.numpy as jnp
from jax import lax
from jax.experimental import pallas as pl
from jax.experimental.pallas import tpu as pltpu
from jax.experimental.pallas import tpu_sc as plsc


@functools.lru_cache(maxsize=None)
def _make_select(N: int, C: int, H: int, W: int, K: int):
    info = plsc.get_sparse_core_info()
    NC, NS = info.num_cores, info.num_subcores
    NW = NC * NS
    PLANES = N * H
    PPW = PLANES // NW
    NJ = K // 16

    mesh = plsc.VectorSubcoreMesh(core_axis_name="c", subcore_axis_name="s")

    @functools.partial(
        pl.kernel,
        mesh=mesh,
        compiler_params=pltpu.CompilerParams(needs_layout_passes=False),
        out_type=jax.ShapeDtypeStruct((N, H, W, K), jnp.float32),
        scratch_types=[
            pltpu.VMEM((K,), jnp.int32),
            pltpu.VMEM((2, W, C), jnp.float32),
            pltpu.VMEM((2, W, K), jnp.float32),
            pltpu.SemaphoreType.DMA,
            pltpu.SemaphoreType.DMA,
            pltpu.SemaphoreType.DMA,
            pltpu.SemaphoreType.DMA,
        ],
    )
    def select_kernel(x_hbm, idx_hbm, out_hbm, idxv, xbuf, obuf,
                      sg0, sg1, sw0, sw1):
        wid = lax.axis_index("s") * NC + lax.axis_index("c")
        pltpu.sync_copy(idx_hbm, idxv)
        p0 = wid * PPW

        def plane_compute(b):
            cvecs = [idxv[pl.ds(jv * 16, 16)] for jv in range(NJ)]
            for w in range(W):
                wvec = jnp.full((16,), w, jnp.int32)
                for jv in range(NJ):
                    v = plsc.load_gather(xbuf.at[b], [wvec, cvecs[jv]])
                    obuf[b, w, pl.ds(jv * 16, 16)] = v

        def gather_start(p, b, sem):
            n = p // H
            h = p % H
            return pltpu.async_copy(x_hbm.at[n, h], xbuf.at[b], sem)

        def write_start(p, b, sem):
            n = p // H
            h = p % H
            return pltpu.async_copy(obuf.at[b], out_hbm.at[n, h], sem)

        def body(i, carry):
            pa = p0 + 2 * i
            pb = pa + 1
            ga = gather_start(pa, 0, sg0)
            gb = gather_start(pb, 1, sg1)
            ga.wait()
            plane_compute(0)
            wa = write_start(pa, 0, sw0)
            gb.wait()
            plane_compute(1)
            wb = write_start(pb, 1, sw1)
            wa.wait()
            wb.wait()
            return carry

        lax.fori_loop(0, PPW // 2, body, 0)

    return select_kernel


def kernel(input_tensor, indexes):
    N, C, H, W = input_tensor.shape
    K = indexes.shape[0]
    if K == C:
        return input_tensor
    x_nhwc = jnp.transpose(input_tensor, (0, 2, 3, 1))
    out_nhwc = _make_select(N, C, H, W, K)(x_nhwc, indexes)
    return jnp.transpose(out_nhwc, (0, 3, 1, 2))


# speedup vs baseline: 5.9894x; 1.9905x over previous
import functools

import jax
import jax.numpy as jnp
from jax import lax
from jax.experimental import pallas as pl
from jax.experimental.pallas import tpu as pltpu
from jax.experimental.pallas import tpu_sc as plsc


@functools.lru_cache(maxsize=None)
def _make_select(N: int, C: int, H: int, W: int, K: int):
    info = plsc.get_sparse_core_info()
    NC, NS = info.num_cores, info.num_subcores
    NW = NC * NS
    PLANES = N * H
    PPW = PLANES // NW
    NJ = K // 16

    mesh = plsc.VectorSubcoreMesh(core_axis_name="c", subcore_axis_name="s")

    @functools.partial(
        pl.kernel,
        mesh=mesh,
        compiler_params=pltpu.CompilerParams(needs_layout_passes=False),
        out_type=jax.ShapeDtypeStruct((N, H, W, K), jnp.float32),
        scratch_types=[
            pltpu.VMEM((K,), jnp.int32),
            pltpu.VMEM((2, W, C), jnp.float32),
            pltpu.VMEM((2, W, K), jnp.float32),
            pltpu.SemaphoreType.DMA,
            pltpu.SemaphoreType.DMA,
            pltpu.SemaphoreType.DMA,
            pltpu.SemaphoreType.DMA,
        ],
    )
    def select_kernel(x_hbm, idx_hbm, out_hbm, idxv, xbuf, obuf,
                      sg0, sg1, sw0, sw1):
        sg = (sg0, sg1)
        sw = (sw0, sw1)
        wid = lax.axis_index("s") * NC + lax.axis_index("c")
        pltpu.sync_copy(idx_hbm, idxv)
        p0 = wid * PPW

        def plane_compute(b):
            cvecs = [idxv[pl.ds(jv * 16, 16)] for jv in range(NJ)]
            for w in range(W):
                wvec = jnp.full((16,), w, jnp.int32)
                vals = [plsc.load_gather(xbuf.at[b], [wvec, cvecs[jv]])
                        for jv in range(NJ)]
                for jv in range(NJ):
                    obuf[b, w, pl.ds(jv * 16, 16)] = vals[jv]

        def gather_start(p, b):
            n = p // H
            h = p % H
            return pltpu.async_copy(x_hbm.at[n, h], xbuf.at[b], sg[b])

        def write_start(p, b):
            n = p // H
            h = p % H
            return pltpu.async_copy(obuf.at[b], out_hbm.at[n, h], sw[b])

        def drain_write(b):
            pltpu.make_async_copy(out_hbm.at[0, 0], obuf.at[b], sw[b]).wait()

        NB = PPW // 2
        gather_start(p0, 0)
        gather_start(p0 + 1, 1)

        def body(i, carry):
            p = p0 + 2 * i
            for b in range(2):
                pltpu.make_async_copy(x_hbm.at[0, 0], xbuf.at[b], sg[b]).wait()

                @pl.when(i > 0)
                def _():
                    drain_write(b)

                plane_compute(b)
                write_start(p + b, b)

                @pl.when(i + 1 < NB)
                def _():
                    gather_start(p + b + 2, b)

            return carry

        lax.fori_loop(0, NB, body, 0)
        drain_write(0)
        drain_write(1)

    return select_kernel


def kernel(input_tensor, indexes):
    N, C, H, W = input_tensor.shape
    K = indexes.shape[0]
    if K == C:
        return input_tensor
    x_nhwc = jnp.transpose(input_tensor, (0, 2, 3, 1))
    out_nhwc = _make_select(N, C, H, W, K)(x_nhwc, indexes)
    return jnp.transpose(out_nhwc, (0, 3, 1, 2))
